# baseline (speedup 1.0000x reference)
import functools

import jax
import jax.numpy as jnp
import numpy as np
from jax import lax
from jax.experimental import pallas as pl
from jax.experimental.pallas import tpu as pltpu

_BS = 5
_B, _C, _H, _W = 32, 192, 56, 56
_HM, _WM = _H - (_BS - 1), _W - (_BS - 1)
_IMGS = _B * _C
_SEED_ELEMS = _IMGS * _HM * _WM
_SEED_ROWS = _SEED_ELEMS // 128

_K0 = 0
_K1 = 42
_K2 = _K0 ^ _K1 ^ 0x1BD11BDA
_KS = (_K0, _K1, _K2)
_ROT = ((13, 15, 26, 6), (17, 29, 16, 24))

_RNG_GRID = 338
_RNG_BLK_ROWS = 384
_RNG_CHUNK = 64
_G1B = 64
_IC1B = 4
_G2 = 64
_IC2 = 2


def _threefry_bits(idx):
    x0 = jnp.full_like(idx, np.uint32(_K0))
    x1 = idx + np.uint32(_K1)
    for g in range(5):
        for r in _ROT[g & 1]:
            x0 = x0 + x1
            x1 = (x1 << r) | (x1 >> (32 - r))
            x1 = x1 ^ x0
        x0 = x0 + np.uint32(_KS[(g + 1) % 3])
        x1 = x1 + np.uint32((_KS[(g + 2) % 3] + g + 1) & 0xFFFFFFFF)
    return x0 ^ x1


def _rng_body(gamma_ref, seed_ref):
    pid = pl.program_id(0)
    gamma = gamma_ref[0, 0]
    it_r = lax.broadcasted_iota(jnp.uint32, (_RNG_CHUNK, 128), 0)
    it_c = lax.broadcasted_iota(jnp.uint32, (_RNG_CHUNK, 128), 1)

    def body(i, _):
        base = ((pid * _RNG_BLK_ROWS + i * _RNG_CHUNK) * 128).astype(jnp.uint32)
        idx = base + it_r * np.uint32(128) + it_c
        bits = _threefry_bits(idx)
        f = lax.bitcast_convert_type(
            (bits >> 9) | np.uint32(0x3F800000), jnp.float32
        )
        u = f - 1.0
        seed_ref[pl.ds(i * _RNG_CHUNK, _RNG_CHUNK), :] = (u < gamma).astype(
            jnp.float32
        )
        return 0

    lax.fori_loop(0, _RNG_BLK_ROWS // _RNG_CHUNK, body, 0)


def _dilate(s):
    n = s.shape[0]
    zc = jnp.zeros((n, _HM, _BS - 1), s.dtype)
    t = jnp.concatenate([zc, s, zc], axis=2)
    zr = jnp.zeros((n, _BS - 1, t.shape[2]), s.dtype)
    z = jnp.concatenate([zr, t, zr], axis=1)
    r = z[:, 0:_H, :]
    for i in range(1, _BS):
        r = jnp.maximum(r, z[:, i : i + _H, :])
    c = r[:, :, 0:_W]
    for j in range(1, _BS):
        c = jnp.maximum(c, r[:, :, j : j + _W])
    return c


def _count_body(seed_ref, cnt_ref):
    def body(i, acc):
        s = seed_ref[pl.ds(i * _IC1B, _IC1B)]
        return acc + jnp.sum(_dilate(s))

    acc = lax.fori_loop(0, _G1B // _IC1B, body, jnp.zeros((1, 1), jnp.float32))
    cnt_ref[0] = acc


def _apply_body(scale_ref, x_ref, seed_ref, o_ref):
    sc = scale_ref[0, 0]

    def body(i, _):
        sl = pl.ds(i * _IC2, _IC2)
        pm = _dilate(seed_ref[sl])
        o_ref[sl] = x_ref[sl] * ((1.0 - pm) * sc)
        return 0

    lax.fori_loop(0, _G2 // _IC2, body, 0)


@functools.partial(jax.jit, static_argnames=())
def kernel(x, gamma):
    gamma2 = jnp.reshape(gamma.astype(jnp.float32), (1, 1))

    seeds_flat = pl.pallas_call(
        _rng_body,
        grid=(_RNG_GRID,),
        in_specs=[
            pl.BlockSpec(memory_space=pltpu.SMEM),
        ],
        out_specs=pl.BlockSpec((_RNG_BLK_ROWS, 128), lambda i: (i, 0)),
        out_shape=jax.ShapeDtypeStruct((_SEED_ROWS, 128), jnp.float32),
    )(gamma2)
    seeds = seeds_flat.reshape(_IMGS, _HM, _WM)

    cnt = pl.pallas_call(
        _count_body,
        grid=(_IMGS // _G1B,),
        in_specs=[pl.BlockSpec((_G1B, _HM, _WM), lambda i: (i, 0, 0))],
        out_specs=pl.BlockSpec((1, 1, 1), lambda i: (i, 0, 0)),
        out_shape=jax.ShapeDtypeStruct((_IMGS // _G1B, 1, 1), jnp.float32),
    )(seeds)

    count_m = float(_B * _C * _H * _W)
    dropped = jnp.sum(cnt)
    scale = count_m / (count_m - dropped)
    scale2 = jnp.reshape(scale, (1, 1))

    xf = x.reshape(_IMGS, _H, _W)
    y = pl.pallas_call(
        _apply_body,
        grid=(_IMGS // _G2,),
        in_specs=[
            pl.BlockSpec(memory_space=pltpu.SMEM),
            pl.BlockSpec((_G2, _H, _W), lambda i: (i, 0, 0)),
            pl.BlockSpec((_G2, _HM, _WM), lambda i: (i, 0, 0)),
        ],
        out_specs=pl.BlockSpec((_G2, _H, _W), lambda i: (i, 0, 0)),
        out_shape=jax.ShapeDtypeStruct((_IMGS, _H, _W), jnp.float32),
    )(scale2, xf, seeds)
    return y.reshape(_B, _C, _H, _W)


# speedup vs baseline: 1.6989x; 1.6989x over previous
import functools

import jax
import jax.numpy as jnp
import numpy as np
from jax import lax
from jax.experimental import pallas as pl
from jax.experimental.pallas import tpu as pltpu

_BS = 5
_B, _C, _H, _W = 32, 192, 56, 56
_HM, _WM = _H - (_BS - 1), _W - (_BS - 1)
_IMGS = _B * _C
_PER_IMG = _HM * _WM
_OUT_ELEMS = _IMGS * _H * _W

_K0 = 0
_K1 = 42
_K2 = _K0 ^ _K1 ^ 0x1BD11BDA
_KS = (_K0, _K1, _K2)
_ROT = ((13, 15, 26, 6), (17, 29, 16, 24))

_GA = 64
_PK = 2
_ICA = 2 * _PK
_NSTEPS = _IMGS // _GA
_GB = 64
_ICB = 2


def _threefry_bits(idx):
    x0 = jnp.full_like(idx, np.uint32(_K0))
    x1 = idx + np.uint32(_K1)
    for g in range(5):
        for r in _ROT[g & 1]:
            x0 = x0 + x1
            x1 = (x1 << r) | (x1 >> (32 - r))
            x1 = x1 ^ x0
        x0 = x0 + np.uint32(_KS[(g + 1) % 3])
        x1 = x1 + np.uint32((_KS[(g + 2) % 3] + g + 1) & 0xFFFFFFFF)
    return x0 ^ x1


def _shift_down(t, k):
    n = t.shape[0]
    z = jnp.zeros((n, k, t.shape[2]), t.dtype)
    return jnp.concatenate([z, t[:, : t.shape[1] - k, :]], axis=1)


def _mask_body(gamma_ref, pm_ref, cnt_ref):
    pid = pl.program_id(0)
    thresh = gamma_ref[0, 0] * np.float32(8388608.0)

    shp = (_PK, _HM, 2 * _WM)
    it_p = lax.broadcasted_iota(jnp.uint32, shp, 0)
    it_r = lax.broadcasted_iota(jnp.uint32, shp, 1)
    it_l = lax.broadcasted_iota(jnp.uint32, shp, 2)
    lhi = (it_l >= _WM).astype(jnp.uint32)
    img = it_p + lhi * np.uint32(_PK)
    col = it_l - lhi * np.uint32(_WM)
    local = img * np.uint32(_PER_IMG) + it_r * np.uint32(_WM) + col

    ii = lax.broadcasted_iota(jnp.int32, (2 * _WM, 2 * _W), 0)
    jj = lax.broadcasted_iota(jnp.int32, (2 * _WM, 2 * _W), 1)
    ib = (ii >= _WM).astype(jnp.int32)
    jb = (jj >= _W).astype(jnp.int32)
    d = (jj - jb * _W) - (ii - ib * _WM)
    q = ((d >= 0) & (d < _BS) & (ib == jb)).astype(jnp.float32)

    zp = jnp.zeros((_PK, _H - _HM, 2 * _WM), jnp.float32)

    def body(i, acc):
        base = ((pid * _GA + i * _ICA) * _PER_IMG).astype(jnp.uint32)
        bits = _threefry_bits(local + base)
        t = (bits >> 9).astype(jnp.float32)
        seed = (t < thresh).astype(jnp.float32)
        sp = jnp.concatenate([seed, zp], axis=1)
        s2 = sp.reshape(_PK * _H, 2 * _WM)
        t1 = lax.dot_general(
            s2, q, (((1,), (0,)), ((), ())),
            preferred_element_type=jnp.float32,
        ).reshape(_PK, _H, 2 * _W)
        th = (t1 > 0.5).astype(jnp.bfloat16)
        m1 = jnp.maximum(th, _shift_down(th, 1))
        m2 = jnp.maximum(m1, _shift_down(m1, 2))
        pmp = jnp.maximum(m2, _shift_down(th, 4))
        pm4 = jnp.concatenate(
            [pmp[:, :, : _W], pmp[:, :, _W :]], axis=0
        )
        pm_ref[pl.ds(i * _ICA, _ICA)] = pm4
        return acc + jnp.sum(pmp.astype(jnp.float32))

    acc = lax.fori_loop(0, _GA // _ICA, body, jnp.zeros((1, 1), jnp.float32))
    cnt_ref[0] = jnp.broadcast_to(acc, (1, 128)) * np.float32(1.0 / 128.0)


def _apply_body(cnt_ref, x_ref, pm_ref, o_ref):
    count_m = np.float32(_OUT_ELEMS)
    dropped = jnp.sum(cnt_ref[...])
    sc = count_m / (count_m - dropped)

    def body(i, _):
        sl = pl.ds(i * _ICB, _ICB)
        drop = pm_ref[sl] > jnp.bfloat16(0.5)
        o_ref[sl] = jnp.where(drop, jnp.float32(0.0), x_ref[sl] * sc)
        return 0

    lax.fori_loop(0, _GB // _ICB, body, 0)


@functools.partial(jax.jit, static_argnames=())
def kernel(x, gamma):
    gamma2 = jnp.reshape(gamma.astype(jnp.float32), (1, 1))

    pm, cnt = pl.pallas_call(
        _mask_body,
        grid=(_NSTEPS,),
        in_specs=[pl.BlockSpec(memory_space=pltpu.SMEM)],
        out_specs=[
            pl.BlockSpec((_GA, _H, _W), lambda i: (i, 0, 0)),
            pl.BlockSpec((1, 1, 128), lambda i: (i, 0, 0)),
        ],
        out_shape=[
            jax.ShapeDtypeStruct((_IMGS, _H, _W), jnp.bfloat16),
            jax.ShapeDtypeStruct((_NSTEPS, 1, 128), jnp.float32),
        ],
    )(gamma2)

    xf = x.reshape(_IMGS, _H, _W)
    y = pl.pallas_call(
        _apply_body,
        grid=(_IMGS // _GB,),
        in_specs=[
            pl.BlockSpec((_NSTEPS, 1, 128), lambda i: (0, 0, 0)),
            pl.BlockSpec((_GB, _H, _W), lambda i: (i, 0, 0)),
            pl.BlockSpec((_GB, _H, _W), lambda i: (i, 0, 0)),
        ],
        out_specs=pl.BlockSpec((_GB, _H, _W), lambda i: (i, 0, 0)),
        out_shape=jax.ShapeDtypeStruct((_IMGS, _H, _W), jnp.float32),
    )(cnt, xf, pm)
    return y.reshape(_B, _C, _H, _W)


# speedup vs baseline: 1.9617x; 1.1547x over previous
import functools

import jax
import jax.numpy as jnp
import numpy as np
from jax import lax
from jax.experimental import pallas as pl
from jax.experimental.pallas import tpu as pltpu

_BS = 5
_B, _C, _H, _W = 32, 192, 56, 56
_HM, _WM = _H - (_BS - 1), _W - (_BS - 1)
_IMGS = _B * _C
_PER_IMG = _HM * _WM
_OUT_ELEMS = _IMGS * _H * _W

_K0 = 0
_K1 = 42
_K2 = _K0 ^ _K1 ^ 0x1BD11BDA
_KS = (_K0, _K1, _K2)
_ROT = ((13, 15, 26, 6), (17, 29, 16, 24))

_GA = 64
_PK = 2
_ICA = 2 * _PK
_NSTEPS = _IMGS // _GA
_GB = 64
_ICB = 4


def _threefry_bits(idx):
    x0 = jnp.full_like(idx, np.uint32(_K0))
    x1 = idx + np.uint32(_K1)
    for g in range(5):
        for r in _ROT[g & 1]:
            x0 = x0 + x1
            x1 = (x1 << r) | (x1 >> (32 - r))
            x1 = x1 ^ x0
        x0 = x0 + np.uint32(_KS[(g + 1) % 3])
        x1 = x1 + np.uint32((_KS[(g + 2) % 3] + g + 1) & 0xFFFFFFFF)
    return x0 ^ x1


def _shift_down(t, k):
    n = t.shape[0]
    z = jnp.zeros((n, k, t.shape[2]), t.dtype)
    return jnp.concatenate([z, t[:, : t.shape[1] - k, :]], axis=1)


def _mask_body(gamma_ref, pm_ref, cnt_ref):
    pid = pl.program_id(0)
    thresh = gamma_ref[0, 0] * np.float32(8388608.0)

    shp = (_PK, _HM, 2 * _WM)
    it_p = lax.broadcasted_iota(jnp.uint32, shp, 0)
    it_r = lax.broadcasted_iota(jnp.uint32, shp, 1)
    it_l = lax.broadcasted_iota(jnp.uint32, shp, 2)
    lhi = (it_l >= _WM).astype(jnp.uint32)
    img = it_p + lhi * np.uint32(_PK)
    col = it_l - lhi * np.uint32(_WM)
    local = img * np.uint32(_PER_IMG) + it_r * np.uint32(_WM) + col

    ii = lax.broadcasted_iota(jnp.int32, (2 * _WM, 2 * _W), 0)
    jj = lax.broadcasted_iota(jnp.int32, (2 * _WM, 2 * _W), 1)
    ib = (ii >= _WM).astype(jnp.int32)
    jb = (jj >= _W).astype(jnp.int32)
    d = (jj - jb * _W) - (ii - ib * _WM)
    q = ((d >= 0) & (d < _BS) & (ib == jb)).astype(jnp.float32)

    zp = jnp.zeros((_PK, _H - _HM, 2 * _WM), jnp.float32)

    def make_sp(i):
        base = ((pid * _GA + i * _ICA) * _PER_IMG).astype(jnp.uint32)
        bits = _threefry_bits(local + base)
        t = (bits >> 9).astype(jnp.float32)
        seed = (t < thresh).astype(jnp.float32)
        return jnp.concatenate([seed, zp], axis=1)

    def body(i, carry):
        acc, sp = carry
        s2 = sp.reshape(_PK * _H, 2 * _WM)
        t1 = lax.dot_general(
            s2, q, (((1,), (0,)), ((), ())),
            preferred_element_type=jnp.float32,
        ).reshape(_PK, _H, 2 * _W)
        sp_next = make_sp(i + 1)
        th = (t1 > 0.5).astype(jnp.bfloat16)
        m1 = jnp.maximum(th, _shift_down(th, 1))
        m2 = jnp.maximum(m1, _shift_down(m1, 2))
        pmp = jnp.maximum(m2, _shift_down(th, 4))
        pm4 = jnp.concatenate(
            [pmp[:, :, : _W], pmp[:, :, _W :]], axis=0
        )
        pm_ref[pl.ds(i * _ICA, _ICA)] = pm4
        return acc + jnp.sum(pmp.astype(jnp.float32)), sp_next

    acc, _ = lax.fori_loop(
        0,
        _GA // _ICA,
        body,
        (jnp.zeros((1, 1), jnp.float32), make_sp(0)),
    )
    cnt_ref[0] = jnp.broadcast_to(acc, (1, 128)) * np.float32(1.0 / 128.0)


def _apply_body(cnt_ref, x_ref, pm_ref, o_ref):
    count_m = np.float32(_OUT_ELEMS)
    dropped = jnp.sum(cnt_ref[...])
    sc = count_m / (count_m - dropped)

    def body(i, _):
        sl = pl.ds(i * _ICB, _ICB)
        drop = pm_ref[sl] > jnp.bfloat16(0.5)
        o_ref[sl] = jnp.where(drop, jnp.float32(0.0), x_ref[sl] * sc)
        return 0

    lax.fori_loop(0, _GB // _ICB, body, 0)


@functools.partial(jax.jit, static_argnames=())
def kernel(x, gamma):
    gamma2 = jnp.reshape(gamma.astype(jnp.float32), (1, 1))

    pm, cnt = pl.pallas_call(
        _mask_body,
        grid=(_NSTEPS,),
        in_specs=[pl.BlockSpec(memory_space=pltpu.SMEM)],
        out_specs=[
            pl.BlockSpec((_GA, _H, _W), lambda i: (i, 0, 0)),
            pl.BlockSpec((1, 1, 128), lambda i: (i, 0, 0)),
        ],
        out_shape=[
            jax.ShapeDtypeStruct((_IMGS, _H, _W), jnp.bfloat16),
            jax.ShapeDtypeStruct((_NSTEPS, 1, 128), jnp.float32),
        ],
    )(gamma2)

    xf = x.reshape(_IMGS, _H, _W)
    y = pl.pallas_call(
        _apply_body,
        grid=(_IMGS // _GB,),
        in_specs=[
            pl.BlockSpec((_NSTEPS, 1, 128), lambda i: (0, 0, 0)),
            pl.BlockSpec((_GB, _H, _W), lambda i: (i, 0, 0)),
            pl.BlockSpec((_GB, _H, _W), lambda i: (i, 0, 0)),
        ],
        out_specs=pl.BlockSpec((_GB, _H, _W), lambda i: (i, 0, 0)),
        out_shape=jax.ShapeDtypeStruct((_IMGS, _H, _W), jnp.float32),
    )(cnt, xf, pm)
    return y.reshape(_B, _C, _H, _W)


# speedup vs baseline: 2.4395x; 1.2435x over previous
import functools

import jax
import jax.numpy as jnp
import numpy as np
from jax import lax
from jax.experimental import pallas as pl
from jax.experimental.pallas import tpu as pltpu

_BS = 5
_B, _C, _H, _W = 32, 192, 56, 56
_HM, _WM = _H - (_BS - 1), _W - (_BS - 1)
_PER_IMG = _HM * _WM
_OUT_ELEMS = _B * _C * _H * _W

_K0 = 0
_K1 = 42
_K2 = _K0 ^ _K1 ^ 0x1BD11BDA
_KS = (_K0, _K1, _K2)
_ROT = ((13, 15, 26, 6), (17, 29, 16, 24))

_ICB = 4


def _threefry_bits(idx):
    x0 = jnp.full_like(idx, np.uint32(_K0))
    x1 = idx + np.uint32(_K1)
    for g in range(5):
        for r in _ROT[g & 1]:
            x0 = x0 + x1
            x1 = (x1 << r) | (x1 >> (32 - r))
            x1 = x1 ^ x0
        x0 = x0 + np.uint32(_KS[(g + 1) % 3])
        x1 = x1 + np.uint32((_KS[(g + 2) % 3] + g + 1) & 0xFFFFFFFF)
    return x0 ^ x1


def _shift_down_w(v, k):
    z = jnp.zeros((k, v.shape[1]), v.dtype)
    return jnp.concatenate([z, v[: v.shape[0] - k]], axis=0)


def _mask_body(gamma_ref, pm_ref, cnt_ref, scr_ref):
    b = pl.program_id(0)
    tu = jnp.ceil(gamma_ref[0, 0] * np.float32(8388608.0)).astype(jnp.uint32)

    it_w = lax.broadcasted_iota(jnp.uint32, (_WM, _C), 0)
    it_c = lax.broadcasted_iota(jnp.uint32, (_WM, _C), 1)
    iota0 = it_c * np.uint32(_PER_IMG) + it_w

    zrow = jnp.zeros((_BS - 1, _W, _C), jnp.float32)
    scr_ref[pl.ds(0, _BS - 1)] = zrow
    scr_ref[pl.ds(_H, _BS - 1)] = zrow

    base_b = np.uint32(_C * _PER_IMG) * b.astype(jnp.uint32)

    def row_body(h, _):
        idx = iota0 + (base_b + h.astype(jnp.uint32) * np.uint32(_WM))
        bits = _threefry_bits(idx)
        seed = ((bits >> 9) < tu).astype(jnp.float32)
        sp = jnp.concatenate(
            [seed, jnp.zeros((_W - _WM, _C), jnp.float32)], axis=0
        )
        m1 = jnp.maximum(sp, _shift_down_w(sp, 1))
        m2 = jnp.maximum(m1, _shift_down_w(m1, 2))
        cw = jnp.maximum(m2, _shift_down_w(sp, 4))
        scr_ref[h + _BS - 1] = cw
        return 0

    lax.fori_loop(0, _HM, row_body, 0)

    def out_body(ho, acc):
        rows = scr_ref[pl.ds(ho, _BS)]
        hm = jnp.max(rows, axis=0)
        pm_ref[0, ho] = (hm > 0.5).astype(jnp.bfloat16)
        return acc + jnp.sum((hm > 0.5).astype(jnp.float32))

    acc = lax.fori_loop(0, _H, out_body, jnp.zeros((1, 1), jnp.float32))
    cnt_ref[0] = jnp.broadcast_to(acc, (1, 128)) * np.float32(1.0 / 128.0)


def _apply_body(cnt_ref, x_ref, pm_ref, o_ref):
    count_m = np.float32(_OUT_ELEMS)
    dropped = jnp.sum(cnt_ref[...])
    sc = count_m / (count_m - dropped)

    def body(i, _):
        sl = pl.ds(i * _ICB, _ICB)
        drop = pm_ref[0, sl] > jnp.bfloat16(0.5)
        o_ref[0, sl] = jnp.where(drop, jnp.float32(0.0), x_ref[0, sl] * sc)
        return 0

    lax.fori_loop(0, _H // _ICB, body, 0)


@functools.partial(jax.jit, static_argnames=())
def kernel(x, gamma):
    gamma2 = jnp.reshape(gamma.astype(jnp.float32), (1, 1))
    xt = jnp.transpose(x, (0, 2, 3, 1))

    pm, cnt = pl.pallas_call(
        _mask_body,
        grid=(_B,),
        in_specs=[pl.BlockSpec(memory_space=pltpu.SMEM)],
        out_specs=[
            pl.BlockSpec((1, _H, _W, _C), lambda i: (i, 0, 0, 0)),
            pl.BlockSpec((1, 1, 128), lambda i: (i, 0, 0)),
        ],
        out_shape=[
            jax.ShapeDtypeStruct((_B, _H, _W, _C), jnp.bfloat16),
            jax.ShapeDtypeStruct((_B, 1, 128), jnp.float32),
        ],
        scratch_shapes=[pltpu.VMEM((_H + _BS - 1, _W, _C), jnp.float32)],
    )(gamma2)

    yt = pl.pallas_call(
        _apply_body,
        grid=(_B,),
        in_specs=[
            pl.BlockSpec((_B, 1, 128), lambda i: (0, 0, 0)),
            pl.BlockSpec((1, _H, _W, _C), lambda i: (i, 0, 0, 0)),
            pl.BlockSpec((1, _H, _W, _C), lambda i: (i, 0, 0, 0)),
        ],
        out_specs=pl.BlockSpec((1, _H, _W, _C), lambda i: (i, 0, 0, 0)),
        out_shape=jax.ShapeDtypeStruct((_B, _H, _W, _C), jnp.float32),
    )(cnt, xt, pm)
    return jnp.transpose(yt, (0, 3, 1, 2))


# speedup vs baseline: 2.4838x; 1.0182x over previous
import functools

import jax
import jax.numpy as jnp
import numpy as np
from jax import lax
from jax.experimental import pallas as pl
from jax.experimental.pallas import tpu as pltpu

_BS = 5
_B, _C, _H, _W = 32, 192, 56, 56
_HM, _WM = _H - (_BS - 1), _W - (_BS - 1)
_PER_IMG = _HM * _WM
_OUT_ELEMS = _B * _C * _H * _W

_K0 = 0
_K1 = 42
_K2 = _K0 ^ _K1 ^ 0x1BD11BDA
_KS = (_K0, _K1, _K2)
_ROT = ((13, 15, 26, 6), (17, 29, 16, 24))

_ICB = 4


def _threefry_bits(idx):
    x0 = jnp.full_like(idx, np.uint32(_K0))
    x1 = idx + np.uint32(_K1)
    for g in range(5):
        for r in _ROT[g & 1]:
            x0 = x0 + x1
            x1 = (x1 << r) | (x1 >> (32 - r))
            x1 = x1 ^ x0
        x0 = x0 + np.uint32(_KS[(g + 1) % 3])
        x1 = x1 + np.uint32((_KS[(g + 2) % 3] + g + 1) & 0xFFFFFFFF)
    return x0 ^ x1


def _shift_down_w(v, k):
    z = jnp.zeros((v.shape[0], k, v.shape[2]), v.dtype)
    return jnp.concatenate([z, v[:, : v.shape[1] - k]], axis=1)


def _mask_body(gamma_ref, pm_ref, cnt_ref, scr_ref):
    b = pl.program_id(0)
    tu = jnp.ceil(gamma_ref[0, 0] * np.float32(8388608.0)).astype(jnp.uint32)

    shp = (2, _WM, _C)
    it_p = lax.broadcasted_iota(jnp.uint32, shp, 0)
    it_w = lax.broadcasted_iota(jnp.uint32, shp, 1)
    it_c = lax.broadcasted_iota(jnp.uint32, shp, 2)
    iota0 = it_c * np.uint32(_PER_IMG) + it_p * np.uint32(_WM) + it_w

    base_b = np.uint32(_C * _PER_IMG) * b.astype(jnp.uint32)
    zpad = jnp.zeros((2, _W - _WM, _C), jnp.float32)

    def make_sp(i):
        idx = iota0 + (base_b + i.astype(jnp.uint32) * np.uint32(2 * _WM))
        bits = _threefry_bits(idx)
        seed = ((bits >> 9) < tu).astype(jnp.float32)
        return jnp.concatenate([seed, zpad], axis=1)

    def row_body(i, sp_prev):
        m1 = jnp.maximum(sp_prev, _shift_down_w(sp_prev, 1))
        m2 = jnp.maximum(m1, _shift_down_w(m1, 2))
        cw = jnp.maximum(m2, _shift_down_w(sp_prev, 4))
        scr_ref[pl.ds(2 * i + 2, 2)] = cw
        return make_sp(i)

    lax.fori_loop(
        0, _HM // 2 + 1, row_body, jnp.zeros((2, _W, _C), jnp.float32)
    )
    zrow = jnp.zeros((_BS - 1, _W, _C), jnp.float32)
    scr_ref[pl.ds(0, _BS - 1)] = zrow
    scr_ref[pl.ds(_H, _BS - 1)] = zrow

    def out_body(ho, acc):
        rows = scr_ref[pl.ds(ho, _BS)]
        hm = jnp.max(rows, axis=0)
        pm_ref[0, ho] = (hm > 0.5).astype(jnp.bfloat16)
        return acc + jnp.sum((hm > 0.5).astype(jnp.float32))

    acc = lax.fori_loop(0, _H, out_body, jnp.zeros((1, 1), jnp.float32))
    cnt_ref[0] = jnp.broadcast_to(acc, (1, 128)) * np.float32(1.0 / 128.0)


def _apply_body(cnt_ref, x_ref, pm_ref, o_ref):
    count_m = np.float32(_OUT_ELEMS)
    dropped = jnp.sum(cnt_ref[...])
    sc = count_m / (count_m - dropped)

    def body(i, _):
        sl = pl.ds(i * _ICB, _ICB)
        drop = pm_ref[0, sl] > jnp.bfloat16(0.5)
        o_ref[0, sl] = jnp.where(drop, jnp.float32(0.0), x_ref[0, sl] * sc)
        return 0

    lax.fori_loop(0, _H // _ICB, body, 0)


@functools.partial(jax.jit, static_argnames=())
def kernel(x, gamma):
    gamma2 = jnp.reshape(gamma.astype(jnp.float32), (1, 1))
    xt = jnp.transpose(x, (0, 2, 3, 1))

    pm, cnt = pl.pallas_call(
        _mask_body,
        grid=(_B,),
        in_specs=[pl.BlockSpec(memory_space=pltpu.SMEM)],
        out_specs=[
            pl.BlockSpec((1, _H, _W, _C), lambda i: (i, 0, 0, 0)),
            pl.BlockSpec((1, 1, 128), lambda i: (i, 0, 0)),
        ],
        out_shape=[
            jax.ShapeDtypeStruct((_B, _H, _W, _C), jnp.bfloat16),
            jax.ShapeDtypeStruct((_B, 1, 128), jnp.float32),
        ],
        scratch_shapes=[pltpu.VMEM((_H + _BS - 1, _W, _C), jnp.float32)],
    )(gamma2)

    yt = pl.pallas_call(
        _apply_body,
        grid=(_B,),
        in_specs=[
            pl.BlockSpec((_B, 1, 128), lambda i: (0, 0, 0)),
            pl.BlockSpec((1, _H, _W, _C), lambda i: (i, 0, 0, 0)),
            pl.BlockSpec((1, _H, _W, _C), lambda i: (i, 0, 0, 0)),
        ],
        out_specs=pl.BlockSpec((1, _H, _W, _C), lambda i: (i, 0, 0, 0)),
        out_shape=jax.ShapeDtypeStruct((_B, _H, _W, _C), jnp.float32),
    )(cnt, xt, pm)
    return jnp.transpose(yt, (0, 3, 1, 2))


# speedup vs baseline: 2.5744x; 1.0365x over previous
import functools

import jax
import jax.numpy as jnp
import numpy as np
from jax import lax
from jax.experimental import pallas as pl
from jax.experimental.pallas import tpu as pltpu

_BS = 5
_B, _C, _H, _W = 32, 192, 56, 56
_HM, _WM = _H - (_BS - 1), _W - (_BS - 1)
_PER_IMG = _HM * _WM
_OUT_ELEMS = _B * _C * _H * _W

_K0 = 0
_K1 = 42
_K2 = _K0 ^ _K1 ^ 0x1BD11BDA
_KS = (_K0, _K1, _K2)
_ROT = ((13, 15, 26, 6), (17, 29, 16, 24))

_ICB = 4


def _threefry_bits(idx):
    x0 = jnp.full_like(idx, np.uint32(_K0))
    x1 = idx + np.uint32(_K1)
    for g in range(5):
        for r in _ROT[g & 1]:
            x0 = x0 + x1
            x1 = (x1 << r) | (x1 >> (32 - r))
            x1 = x1 ^ x0
        x0 = x0 + np.uint32(_KS[(g + 1) % 3])
        x1 = x1 + np.uint32((_KS[(g + 2) % 3] + g + 1) & 0xFFFFFFFF)
    return x0 ^ x1


def _shift_down_w(v, k):
    z = jnp.zeros((k, v.shape[1]), v.dtype)
    return jnp.concatenate([z, v[: v.shape[0] - k]], axis=0)


def _mask_body(gamma_ref, pm_ref, cnt_ref, scr_ref):
    b = pl.program_id(0)
    tu = jnp.ceil(gamma_ref[0, 0] * np.float32(8388608.0)).astype(jnp.uint32)

    it_w = lax.broadcasted_iota(jnp.uint32, (_WM, _C), 0)
    it_c = lax.broadcasted_iota(jnp.uint32, (_WM, _C), 1)
    base_b = np.uint32(_C * _PER_IMG) * b.astype(jnp.uint32)
    idx0 = it_c * np.uint32(_PER_IMG) + it_w + base_b

    zpad = jnp.zeros((_W - _WM, _C), jnp.bfloat16)

    def row_body(i, carry):
        idx, sp_prev = carry
        m1 = jnp.maximum(sp_prev, _shift_down_w(sp_prev, 1))
        m2 = jnp.maximum(m1, _shift_down_w(m1, 2))
        cw = jnp.maximum(m2, _shift_down_w(sp_prev, 4))
        scr_ref[i + _BS - 2] = cw
        bits = _threefry_bits(idx)
        seed = ((bits >> 9) < tu).astype(jnp.bfloat16)
        sp = jnp.concatenate([seed, zpad], axis=0)
        return idx + np.uint32(_WM), sp

    lax.fori_loop(
        0,
        _HM + 1,
        row_body,
        (idx0, jnp.zeros((_W, _C), jnp.bfloat16)),
    )
    zrow = jnp.zeros((_BS - 1, _W, _C), jnp.bfloat16)
    scr_ref[pl.ds(0, _BS - 1)] = zrow
    scr_ref[pl.ds(_H, _BS - 1)] = zrow

    def out_body(ho, acc):
        rows = scr_ref[pl.ds(ho, _BS)]
        hm = jnp.max(rows, axis=0)
        drop = hm > jnp.bfloat16(0.5)
        pm_ref[0, ho] = drop.astype(jnp.bfloat16)
        return acc + jnp.sum(drop.astype(jnp.float32))

    acc = lax.fori_loop(0, _H, out_body, jnp.zeros((1, 1), jnp.float32))
    cnt_ref[0] = jnp.broadcast_to(acc, (1, 128)) * np.float32(1.0 / 128.0)


def _apply_body(cnt_ref, x_ref, pm_ref, o_ref):
    count_m = np.float32(_OUT_ELEMS)
    dropped = jnp.sum(cnt_ref[...])
    sc = count_m / (count_m - dropped)

    def body(i, _):
        sl = pl.ds(i * _ICB, _ICB)
        drop = pm_ref[0, sl] > jnp.bfloat16(0.5)
        o_ref[0, sl] = jnp.where(drop, jnp.float32(0.0), x_ref[0, sl] * sc)
        return 0

    lax.fori_loop(0, _H // _ICB, body, 0)


@functools.partial(jax.jit, static_argnames=())
def kernel(x, gamma):
    gamma2 = jnp.reshape(gamma.astype(jnp.float32), (1, 1))
    xt = jnp.transpose(x, (0, 2, 3, 1))

    pm, cnt = pl.pallas_call(
        _mask_body,
        grid=(_B,),
        in_specs=[pl.BlockSpec(memory_space=pltpu.SMEM)],
        out_specs=[
            pl.BlockSpec((1, _H, _W, _C), lambda i: (i, 0, 0, 0)),
            pl.BlockSpec((1, 1, 128), lambda i: (i, 0, 0)),
        ],
        out_shape=[
            jax.ShapeDtypeStruct((_B, _H, _W, _C), jnp.bfloat16),
            jax.ShapeDtypeStruct((_B, 1, 128), jnp.float32),
        ],
        scratch_shapes=[pltpu.VMEM((_H + _BS - 1, _W, _C), jnp.bfloat16)],
    )(gamma2)

    yt = pl.pallas_call(
        _apply_body,
        grid=(_B,),
        in_specs=[
            pl.BlockSpec((_B, 1, 128), lambda i: (0, 0, 0)),
            pl.BlockSpec((1, _H, _W, _C), lambda i: (i, 0, 0, 0)),
            pl.BlockSpec((1, _H, _W, _C), lambda i: (i, 0, 0, 0)),
        ],
        out_specs=pl.BlockSpec((1, _H, _W, _C), lambda i: (i, 0, 0, 0)),
        out_shape=jax.ShapeDtypeStruct((_B, _H, _W, _C), jnp.float32),
    )(cnt, xt, pm)
    return jnp.transpose(yt, (0, 3, 1, 2))


# speedup vs baseline: 3.0231x; 1.1743x over previous
import functools

import jax
import jax.numpy as jnp
import numpy as np
from jax import lax
from jax.experimental import pallas as pl
from jax.experimental.pallas import tpu as pltpu

_BS = 5
_B, _C, _H, _W = 32, 192, 56, 56
_HM, _WM = _H - (_BS - 1), _W - (_BS - 1)
_PER_IMG = _HM * _WM
_OUT_ELEMS = _B * _C * _H * _W

_K0 = 0
_K1 = 42
_K2 = _K0 ^ _K1 ^ 0x1BD11BDA
_KS = (_K0, _K1, _K2)
_ROT = ((13, 15, 26, 6), (17, 29, 16, 24))

_ICB = 4


def _threefry_bits(idx):
    x0 = jnp.full_like(idx, np.uint32(_K0))
    x1 = idx + np.uint32(_K1)
    for g in range(5):
        for r in _ROT[g & 1]:
            x0 = x0 + x1
            x1 = (x1 << r) | (x1 >> (32 - r))
            x1 = x1 ^ x0
        x0 = x0 + np.uint32(_KS[(g + 1) % 3])
        x1 = x1 + np.uint32((_KS[(g + 2) % 3] + g + 1) & 0xFFFFFFFF)
    return x0 ^ x1


def _shift_down_w(v, k):
    z = jnp.zeros((k, v.shape[1]), v.dtype)
    return jnp.concatenate([z, v[: v.shape[0] - k]], axis=0)


def _mask_body(gamma_ref, pm_ref, cnt_ref, scr_ref):
    b = pl.program_id(0)
    tu = jnp.ceil(gamma_ref[0, 0] * np.float32(8388608.0)).astype(jnp.uint32)

    it_w = lax.broadcasted_iota(jnp.uint32, (_WM, _C), 0)
    it_c = lax.broadcasted_iota(jnp.uint32, (_WM, _C), 1)
    base_b = np.uint32(_C * _PER_IMG) * b.astype(jnp.uint32)
    idx0 = it_c * np.uint32(_PER_IMG) + it_w + base_b

    zpad = jnp.zeros((_W - _WM, _C), jnp.bfloat16)

    def row_body(i, carry):
        idx, sp_prev = carry
        m1 = jnp.maximum(sp_prev, _shift_down_w(sp_prev, 1))
        m2 = jnp.maximum(m1, _shift_down_w(m1, 2))
        cw = jnp.maximum(m2, _shift_down_w(sp_prev, 4))
        scr_ref[i + _BS - 2] = cw
        bits = _threefry_bits(idx)
        seed = ((bits >> 9) < tu).astype(jnp.bfloat16)
        sp = jnp.concatenate([seed, zpad], axis=0)
        return idx + np.uint32(_WM), sp

    lax.fori_loop(
        0,
        _HM + 1,
        row_body,
        (idx0, jnp.zeros((_W, _C), jnp.bfloat16)),
    )
    zrow = jnp.zeros((_BS - 1, _W, _C), jnp.bfloat16)
    scr_ref[pl.ds(0, _BS - 1)] = zrow
    scr_ref[pl.ds(_H, _BS - 1)] = zrow

    def out_body(o, acc):
        rows = scr_ref[pl.ds(2 * o, _BS + 1)]
        shared = jnp.max(rows[1:_BS], axis=0)
        hm0 = jnp.maximum(shared, rows[0])
        hm1 = jnp.maximum(shared, rows[_BS])
        drop = jnp.stack([hm0, hm1], axis=0) > jnp.bfloat16(0.5)
        pm_ref[0, pl.ds(2 * o, 2)] = drop.astype(jnp.bfloat16)
        return acc + jnp.sum(drop.astype(jnp.float32))

    acc = lax.fori_loop(0, _H // 2, out_body, jnp.zeros((1, 1), jnp.float32))
    cnt_ref[0] = jnp.broadcast_to(acc, (1, 128)) * np.float32(1.0 / 128.0)


def _apply_body(cnt_ref, x_ref, pm_ref, o_ref):
    count_m = np.float32(_OUT_ELEMS)
    dropped = jnp.sum(cnt_ref[...])
    sc = count_m / (count_m - dropped)

    def body(i, _):
        sl = pl.ds(i * _ICB, _ICB)
        drop = pm_ref[0, sl] > jnp.bfloat16(0.5)
        o_ref[0, sl] = jnp.where(drop, jnp.float32(0.0), x_ref[0, sl] * sc)
        return 0

    lax.fori_loop(0, _H // _ICB, body, 0)


@functools.partial(jax.jit, static_argnames=())
def kernel(x, gamma):
    gamma2 = jnp.reshape(gamma.astype(jnp.float32), (1, 1))
    xt = jnp.transpose(x, (0, 2, 3, 1))

    pm, cnt = pl.pallas_call(
        _mask_body,
        grid=(_B,),
        in_specs=[pl.BlockSpec(memory_space=pltpu.SMEM)],
        out_specs=[
            pl.BlockSpec((1, _H, _W, _C), lambda i: (i, 0, 0, 0)),
            pl.BlockSpec((1, 1, 128), lambda i: (i, 0, 0)),
        ],
        out_shape=[
            jax.ShapeDtypeStruct((_B, _H, _W, _C), jnp.bfloat16),
            jax.ShapeDtypeStruct((_B, 1, 128), jnp.float32),
        ],
        scratch_shapes=[pltpu.VMEM((_H + _BS - 1, _W, _C), jnp.bfloat16)],
    )(gamma2)

    yt = pl.pallas_call(
        _apply_body,
        grid=(_B,),
        in_specs=[
            pl.BlockSpec((_B, 1, 128), lambda i: (0, 0, 0)),
            pl.BlockSpec((1, _H, _W, _C), lambda i: (i, 0, 0, 0)),
            pl.BlockSpec((1, _H, _W, _C), lambda i: (i, 0, 0, 0)),
        ],
        out_specs=pl.BlockSpec((1, _H, _W, _C), lambda i: (i, 0, 0, 0)),
        out_shape=jax.ShapeDtypeStruct((_B, _H, _W, _C), jnp.float32),
    )(cnt, xt, pm)
    return jnp.transpose(yt, (0, 3, 1, 2))


# speedup vs baseline: 3.2805x; 1.0851x over previous
import functools

import jax
import jax.numpy as jnp
import numpy as np
from jax import lax
from jax.experimental import pallas as pl
from jax.experimental.pallas import tpu as pltpu

_BS = 5
_B, _C, _H, _W = 32, 192, 56, 56
_HM, _WM = _H - (_BS - 1), _W - (_BS - 1)
_PER_IMG = _HM * _WM
_OUT_ELEMS = _B * _C * _H * _W

_K0 = 0
_K1 = 42
_K2 = _K0 ^ _K1 ^ 0x1BD11BDA
_KS = (_K0, _K1, _K2)
_ROT = ((13, 15, 26, 6), (17, 29, 16, 24))

_ICB = 4


def _threefry_bits(idx):
    x0 = jnp.full_like(idx, np.uint32(_K0))
    x1 = idx + np.uint32(_K1)
    for g in range(5):
        for r in _ROT[g & 1]:
            x0 = x0 + x1
            x1 = (x1 << r) | (x1 >> (32 - r))
            x1 = x1 ^ x0
        x0 = x0 + np.uint32(_KS[(g + 1) % 3])
        x1 = x1 + np.uint32((_KS[(g + 2) % 3] + g + 1) & 0xFFFFFFFF)
    return x0 ^ x1


def _shift_down_w(v, k):
    z = jnp.zeros((k, v.shape[1]), v.dtype)
    return jnp.concatenate([z, v[: v.shape[0] - k]], axis=0)


def _mask_body(gamma_ref, pm_ref, cnt_ref, scr_ref):
    b = pl.program_id(0)
    tu = jnp.ceil(gamma_ref[0, 0] * np.float32(8388608.0)).astype(jnp.uint32)

    it_w = lax.broadcasted_iota(jnp.uint32, (_WM, _C), 0)
    it_c = lax.broadcasted_iota(jnp.uint32, (_WM, _C), 1)
    base_b = np.uint32(_C * _PER_IMG) * b.astype(jnp.uint32)
    idx0 = it_c * np.uint32(_PER_IMG) + it_w + base_b

    zpad = jnp.zeros((_W - _WM, _C), jnp.bfloat16)

    def row_body(i, carry):
        idx, sp_prev = carry
        m1 = jnp.maximum(sp_prev, _shift_down_w(sp_prev, 1))
        m2 = jnp.maximum(m1, _shift_down_w(m1, 2))
        cw = jnp.maximum(m2, _shift_down_w(sp_prev, 4))
        scr_ref[i + _BS - 2] = cw
        bits = _threefry_bits(idx)
        seed = ((bits >> 9) < tu).astype(jnp.bfloat16)
        sp = jnp.concatenate([seed, zpad], axis=0)
        return idx + np.uint32(_WM), sp

    lax.fori_loop(
        0,
        _HM + 1,
        row_body,
        (idx0, jnp.zeros((_W, _C), jnp.bfloat16)),
    )
    zrow = jnp.zeros((_BS - 1, _W, _C), jnp.bfloat16)
    scr_ref[pl.ds(0, _BS - 1)] = zrow
    scr_ref[pl.ds(_H, _BS - 1)] = zrow

    def out_body(o, acc):
        rows = scr_ref[pl.ds(4 * o, _BS + 3)]
        p = jnp.maximum(rows[:7], rows[1:])
        q = jnp.maximum(p[:5], p[2:])
        hm = jnp.maximum(q[:4], rows[4:])
        drop = hm > jnp.bfloat16(0.5)
        pm_ref[0, pl.ds(4 * o, 4)] = drop.astype(jnp.bfloat16)
        return acc + jnp.sum(drop.astype(jnp.float32))

    acc = lax.fori_loop(0, _H // 4, out_body, jnp.zeros((1, 1), jnp.float32))
    cnt_ref[0] = jnp.broadcast_to(acc, (1, 128)) * np.float32(1.0 / 128.0)


def _apply_body(cnt_ref, x_ref, pm_ref, o_ref):
    count_m = np.float32(_OUT_ELEMS)
    dropped = jnp.sum(cnt_ref[...])
    sc = count_m / (count_m - dropped)

    def body(i, _):
        sl = pl.ds(i * _ICB, _ICB)
        drop = pm_ref[0, sl] > jnp.bfloat16(0.5)
        o_ref[0, sl] = jnp.where(drop, jnp.float32(0.0), x_ref[0, sl] * sc)
        return 0

    lax.fori_loop(0, _H // _ICB, body, 0)


@functools.partial(jax.jit, static_argnames=())
def kernel(x, gamma):
    gamma2 = jnp.reshape(gamma.astype(jnp.float32), (1, 1))
    xt = jnp.transpose(x, (0, 2, 3, 1))

    pm, cnt = pl.pallas_call(
        _mask_body,
        grid=(_B,),
        in_specs=[pl.BlockSpec(memory_space=pltpu.SMEM)],
        out_specs=[
            pl.BlockSpec((1, _H, _W, _C), lambda i: (i, 0, 0, 0)),
            pl.BlockSpec((1, 1, 128), lambda i: (i, 0, 0)),
        ],
        out_shape=[
            jax.ShapeDtypeStruct((_B, _H, _W, _C), jnp.bfloat16),
            jax.ShapeDtypeStruct((_B, 1, 128), jnp.float32),
        ],
        scratch_shapes=[pltpu.VMEM((_H + _BS - 1, _W, _C), jnp.bfloat16)],
    )(gamma2)

    yt = pl.pallas_call(
        _apply_body,
        grid=(_B,),
        in_specs=[
            pl.BlockSpec((_B, 1, 128), lambda i: (0, 0, 0)),
            pl.BlockSpec((1, _H, _W, _C), lambda i: (i, 0, 0, 0)),
            pl.BlockSpec((1, _H, _W, _C), lambda i: (i, 0, 0, 0)),
        ],
        out_specs=pl.BlockSpec((1, _H, _W, _C), lambda i: (i, 0, 0, 0)),
        out_shape=jax.ShapeDtypeStruct((_B, _H, _W, _C), jnp.float32),
    )(cnt, xt, pm)
    return jnp.transpose(yt, (0, 3, 1, 2))
